# Initial kernel scaffold; baseline (speedup 1.0000x reference)
import jax
import jax.numpy as jnp
from jax.experimental import pallas as pl


def kernel(inp, ln_g, ln_b, W_qkv, a_kW1, a_kb1, a_kW2, a_kb2, a_vW1, a_vb1, a_vW2, a_vb2, b_kW1, b_kb1, b_kW2, b_kb2, b_vW1, b_vb1, b_vW2, b_vb2, W_gate, b_gate, W_out):
    raise NotImplementedError("write your pallas kernel here")


# speedup vs baseline: 1.5198x; 1.5198x over previous
import math
import jax
import jax.numpy as jnp
from jax.experimental import pallas as pl
from jax.experimental.pallas import tpu as pltpu

B, L, E, H = 1, 2048, 1024, 16
DH = E // H
LORA = DH // 2
RT = 256
NEG = -1e30

_DN_T = (((1,), (1,)), ((), ()))
_DN_N = (((1,), (0,)), ((), ()))


def _dot_t(a, b):
    return jax.lax.dot_general(a, b, _DN_T, preferred_element_type=jnp.float32)


def _dot_n(a, b):
    return jax.lax.dot_general(a, b, _DN_N, preferred_element_type=jnp.float32)


def _layernorm(x, g, b):
    mu = jnp.mean(x, axis=-1, keepdims=True)
    var = jnp.mean((x - mu) ** 2, axis=-1, keepdims=True)
    return (x - mu) / jnp.sqrt(var + 1e-5) * g + b


def _softmax(logits):
    m = jnp.max(logits, axis=-1, keepdims=True)
    p = jnp.exp(logits - m)
    return p / jnp.sum(p, axis=-1, keepdims=True)


def _pre_kernel(inp_ref, g_ref, b_ref, wqkv_ref, wg_ref, bg_ref,
                qkv_ref, gate_ref):
    x = _layernorm(inp_ref[...], g_ref[...], b_ref[...])
    qkv_ref[...] = _dot_t(x, wqkv_ref[...])
    gate_ref[...] = jax.nn.sigmoid(_dot_t(x, wg_ref[...]) + bg_ref[...])


def _pre(inp, ln_g, ln_b, W_qkv, W_gate, b_gate):
    return pl.pallas_call(
        _pre_kernel,
        grid=(L // RT,),
        in_specs=[
            pl.BlockSpec((RT, E), lambda i: (i, 0)),
            pl.BlockSpec((1, E), lambda i: (0, 0)),
            pl.BlockSpec((1, E), lambda i: (0, 0)),
            pl.BlockSpec((3 * E, E), lambda i: (0, 0)),
            pl.BlockSpec((3 * H, E), lambda i: (0, 0)),
            pl.BlockSpec((1, 3 * H), lambda i: (0, 0)),
        ],
        out_specs=[
            pl.BlockSpec((RT, 3 * E), lambda i: (i, 0)),
            pl.BlockSpec((RT, 3 * H), lambda i: (i, 0)),
        ],
        out_shape=[
            jax.ShapeDtypeStruct((L, 3 * E), jnp.float32),
            jax.ShapeDtypeStruct((L, 3 * H), jnp.float32),
        ],
    )(inp, ln_g.reshape(1, E), ln_b.reshape(1, E), W_qkv, W_gate,
      b_gate.reshape(1, 3 * H))


def _comp_scale(q1, q2, k_ref, v_ref, kb, vb, wk1, bk1, wk2, bk2,
                wv1, bv1, wv2, bv2, blk, topk):
    nblk = L // blk
    kc = _dot_t(jax.nn.sigmoid(1.702 * (_dot_t(kb, wk1) + bk1))
                * (_dot_t(kb, wk1) + bk1), wk2) + bk2
    vc = _dot_t(jax.nn.sigmoid(1.702 * (_dot_t(vb, wv1) + bv1))
                * (_dot_t(vb, wv1) + bv1), wv2) + bv2
    grp = jax.lax.broadcasted_iota(jnp.int32, (L, nblk), 0) // blk
    col = jax.lax.broadcasted_iota(jnp.int32, (L, nblk), 1)
    mask = jnp.where(col <= grp, 0.0, NEG)
    logits = _dot_t(q1, kc) / math.sqrt(LORA) + mask
    w = _softmax(logits)
    comp_out = _dot_n(w, vc)
    scores = jnp.sum(w, axis=0, keepdims=True)
    iota = jax.lax.broadcasted_iota(jnp.int32, (1, nblk), 1)
    qgrp = jax.lax.broadcasted_iota(jnp.int32, (L, 1), 0) // blk
    kparts, vparts, mparts = [], [], []
    sc = scores
    for _ in range(topk):
        mval = jnp.max(sc)
        ind = jnp.min(jnp.where(sc == mval, iota, nblk))
        krows = k_ref[0, pl.ds(ind * blk, blk), :][:, LORA:]
        vrows = v_ref[0, pl.ds(ind * blk, blk), :][:, LORA:]
        kparts.append(krows)
        vparts.append(vrows)
        mparts.append(jnp.broadcast_to(
            jnp.where(ind <= qgrp, 0.0, NEG), (L, blk)))
        sc = jnp.where(iota == ind, NEG, sc)
    ksel = jnp.concatenate(kparts, axis=0)
    vsel = jnp.concatenate(vparts, axis=0)
    selmask = jnp.concatenate(mparts, axis=1)
    logits2 = _dot_t(q2, ksel) / math.sqrt(LORA) + selmask
    w2 = _softmax(logits2)
    slc_out = _dot_n(w2, vsel)
    return comp_out, slc_out


def _nsa_kernel(q_ref, k_ref, v_ref, kba_ref, vba_ref, kbb_ref, vbb_ref,
                akw1, akb1, akw2, akb2, avw1, avb1, avw2, avb2,
                bkw1, bkb1, bkw2, bkb2, bvw1, bvb1, bvw2, bvb2,
                ca_ref, sa_ref, cb_ref, sb_ref):
    q = q_ref[0]
    q1 = q[:, :LORA]
    q2 = q[:, LORA:]
    ca, sa = _comp_scale(q1, q2, k_ref, v_ref, kba_ref[0], vba_ref[0],
                         akw1[...], akb1[...], akw2[...], akb2[...],
                         avw1[...], avb1[...], avw2[...], avb2[...], 32, 2)
    cb, sb = _comp_scale(q1, q2, k_ref, v_ref, kbb_ref[0], vbb_ref[0],
                         bkw1[...], bkb1[...], bkw2[...], bkb2[...],
                         bvw1[...], bvb1[...], bvw2[...], bvb2[...], 8, 8)
    ca_ref[0] = ca
    sa_ref[0] = sa
    cb_ref[0] = cb
    sb_ref[0] = sb


def _nsa(q, k, v, kba, vba, kbb, vbb, wts):
    def head_spec(n, d):
        return pl.BlockSpec((1, n, d), lambda h: (h, 0, 0))

    def full_spec(shape):
        nd = len(shape)
        return pl.BlockSpec(shape, lambda h, _nd=nd: (0,) * _nd)

    in_specs = [
        head_spec(L, DH), head_spec(L, DH), head_spec(L, DH),
        head_spec(64, 32 * LORA), head_spec(64, 32 * LORA),
        head_spec(256, 8 * LORA), head_spec(256, 8 * LORA),
    ] + [full_spec(w.shape) for w in wts]
    out_spec = head_spec(L, LORA)
    return pl.pallas_call(
        _nsa_kernel,
        grid=(H,),
        in_specs=in_specs,
        out_specs=[out_spec] * 4,
        out_shape=[jax.ShapeDtypeStruct((H, L, LORA), jnp.float32)] * 4,
    )(q, k, v, kba, vba, kbb, vbb, *wts)


def _glob_kernel(q_ref, k_ref, v_ref, o_ref):
    i = pl.program_id(1)
    q = q_ref[0]
    logits = _dot_t(q, k_ref[0]) / math.sqrt(DH)
    row = jax.lax.broadcasted_iota(jnp.int32, (RT, L), 0) + i * RT
    col = jax.lax.broadcasted_iota(jnp.int32, (RT, L), 1)
    logits = jnp.where(col <= row, logits, NEG)
    w = _softmax(logits)
    o_ref[0] = _dot_n(w, v_ref[0])


def _glob(q, k, v):
    return pl.pallas_call(
        _glob_kernel,
        grid=(H, L // RT),
        in_specs=[
            pl.BlockSpec((1, RT, DH), lambda h, i: (h, i, 0)),
            pl.BlockSpec((1, L, DH), lambda h, i: (h, 0, 0)),
            pl.BlockSpec((1, L, DH), lambda h, i: (h, 0, 0)),
        ],
        out_specs=pl.BlockSpec((1, RT, DH), lambda h, i: (h, i, 0)),
        out_shape=jax.ShapeDtypeStruct((H, L, DH), jnp.float32),
    )(q, k, v)


def _merge_kernel(ca_ref, sa_ref, cb_ref, sb_ref, glb_ref, gate_ref,
                  g_ref, b_ref, wout_ref, o_ref):
    gt = gate_ref[...]
    pieces = []
    for h in range(H):
        g0 = gt[:, 3 * h:3 * h + 1]
        g1 = gt[:, 3 * h + 1:3 * h + 2]
        g2 = gt[:, 3 * h + 2:3 * h + 3]
        o1 = jnp.concatenate([ca_ref[h], sa_ref[h]], axis=1)
        o2 = jnp.concatenate([cb_ref[h], sb_ref[h]], axis=1)
        pieces.append(g0 * o1 + g1 * o2 + g2 * glb_ref[h])
    y = jnp.concatenate(pieces, axis=1)
    y = _layernorm(y, g_ref[...], b_ref[...])
    o_ref[...] = _dot_t(y, wout_ref[...])


def _merge(ca, sa, cb, sb, glb, gate, ln_g, ln_b, W_out):
    return pl.pallas_call(
        _merge_kernel,
        grid=(L // RT,),
        in_specs=[
            pl.BlockSpec((H, RT, LORA), lambda i: (0, i, 0)),
            pl.BlockSpec((H, RT, LORA), lambda i: (0, i, 0)),
            pl.BlockSpec((H, RT, LORA), lambda i: (0, i, 0)),
            pl.BlockSpec((H, RT, LORA), lambda i: (0, i, 0)),
            pl.BlockSpec((H, RT, DH), lambda i: (0, i, 0)),
            pl.BlockSpec((RT, 3 * H), lambda i: (i, 0)),
            pl.BlockSpec((1, E), lambda i: (0, 0)),
            pl.BlockSpec((1, E), lambda i: (0, 0)),
            pl.BlockSpec((E, E), lambda i: (0, 0)),
        ],
        out_specs=pl.BlockSpec((RT, E), lambda i: (i, 0)),
        out_shape=jax.ShapeDtypeStruct((L, E), jnp.float32),
    )(ca, sa, cb, sb, glb, gate, ln_g.reshape(1, E), ln_b.reshape(1, E),
      W_out)


def kernel(inp, ln_g, ln_b, W_qkv,
           a_kW1, a_kb1, a_kW2, a_kb2, a_vW1, a_vb1, a_vW2, a_vb2,
           b_kW1, b_kb1, b_kW2, b_kb2, b_vW1, b_vb1, b_vW2, b_vb2,
           W_gate, b_gate, W_out):
    qkv, gate = _pre(inp[0], ln_g, ln_b, W_qkv, W_gate, b_gate)
    qh = qkv[:, :E].reshape(L, H, DH).transpose(1, 0, 2)
    kh = qkv[:, E:2 * E].reshape(L, H, DH).transpose(1, 0, 2)
    vh = qkv[:, 2 * E:].reshape(L, H, DH).transpose(1, 0, 2)
    k1 = kh[:, :, :LORA]
    v1 = vh[:, :, :LORA]
    kba = k1.reshape(H, 64, 32 * LORA)
    vba = v1.reshape(H, 64, 32 * LORA)
    kbb = k1.reshape(H, 256, 8 * LORA)
    vbb = v1.reshape(H, 256, 8 * LORA)
    wts = (a_kW1, a_kb1.reshape(1, LORA), a_kW2, a_kb2.reshape(1, LORA),
           a_vW1, a_vb1.reshape(1, LORA), a_vW2, a_vb2.reshape(1, LORA),
           b_kW1, b_kb1.reshape(1, LORA), b_kW2, b_kb2.reshape(1, LORA),
           b_vW1, b_vb1.reshape(1, LORA), b_vW2, b_vb2.reshape(1, LORA))
    ca, sa, cb, sb = _nsa(qh, kh, vh, kba, vba, kbb, vbb, wts)
    glb = _glob(qh, kh, vh)
    out = _merge(ca, sa, cb, sb, glb, gate, ln_g, ln_b, W_out)
    return out.reshape(B, L, E)
